# baseline (speedup 1.0000x reference)
import functools

import jax
import jax.numpy as jnp
from jax import lax
from jax.experimental import pallas as pl
from jax.experimental.pallas import tpu as pltpu
from jax.experimental.pallas import tpu_sc as plsc

B = 16384
CTX = 20
NNEG = 5
NU = NNEG + 1
D = 64
LANES = 16

NC, NS = 2, 16
NW = NC * NS
BPW = B // NW
CB = 16
NCHUNK = BPW // CB
G = D // LANES


VSLICES = ((0, 128), (128, 128), (256, 64))


def _sc_body(ctx_idx, u_idx, u_w, v_w, pos_out, neg_out,
             ctx_v, uidx_v, vrows, urows,
             prodbuf, posbuf, negbuf, sem):
    wid = lax.axis_index("s") * NC + lax.axis_index("c")
    base = wid * BPW

    lane = lax.iota(jnp.int32, LANES)
    shuf_idx = [lane ^ (1 << s) for s in range(4)]
    shuf_msk = [(lane & (1 << s)) == 0 for s in range(4)]
    bitrev = [0, 8, 4, 12, 2, 10, 6, 14, 1, 9, 5, 13, 3, 11, 7, 15]

    def _lane_sums(vecs):
        vs = vecs
        for s in range(3, -1, -1):
            idx, m = shuf_idx[s], shuf_msk[s]
            nxt = []
            for a, b in zip(vs[0::2], vs[1::2]):
                ga = a.at[idx].get(mode="promise_in_bounds")
                gb = b.at[idx].get(mode="promise_in_bounds")
                nxt.append(jnp.where(m, a, gb) + jnp.where(m, ga, b))
            vs = nxt
        return vs[0]

    def chunk(c, carry):
        b0 = base + c * CB
        pltpu.sync_copy(ctx_idx.at[pl.ds(b0 * CTX, CB * CTX)], ctx_v)
        pltpu.sync_copy(u_idx.at[pl.ds(b0 * NU, CB * NU)], uidx_v)
        cps = []
        for s0, sl in ((0, 128), (128, 128), (256, 64)):
            cps.append(pltpu.async_copy(
                v_w.at[ctx_v.at[pl.ds(s0, sl)]], vrows.at[pl.ds(s0, sl)], sem))
        cps.append(pltpu.async_copy(u_w.at[uidx_v], urows, sem))
        for cp in cps:
            cp.wait()
        for i in range(CB):
            vm = []
            for g in range(G):
                acc = vrows[i * CTX, pl.ds(g * LANES, LANES)]
                for j in range(1, CTX):
                    acc = acc + vrows[i * CTX + j, pl.ds(g * LANES, LANES)]
                vm.append(acc)
            for k in range(NU):
                prod = vm[0] * urows[i * NU + k, pl.ds(0, LANES)]
                for g in range(1, G):
                    prod = prod + vm[g] * urows[i * NU + k,
                                                pl.ds(g * LANES, LANES)]
                prodbuf[pl.ds((k * CB + i) * LANES, LANES)] = prod
        for k in range(NU):
            vecs = [prodbuf[pl.ds((k * CB + bitrev[j]) * LANES, LANES)]
                    for j in range(CB)]
            logits = _lane_sums(vecs) * (1.0 / CTX)
            if k == 0:
                posbuf[pl.ds(c * CB, CB)] = logits
            else:
                negbuf[pl.ds((k - 1) * BPW + c * CB, CB)] = logits
        return carry

    lax.fori_loop(0, NCHUNK, chunk, 0)
    pltpu.sync_copy(posbuf, pos_out.at[pl.ds(base, BPW)])
    for k in range(NNEG):
        pltpu.sync_copy(negbuf.at[pl.ds(k * BPW, BPW)],
                        neg_out.at[pl.ds(k * B + base, BPW)])


@functools.cache
def _sc_call():
    mesh = plsc.VectorSubcoreMesh(core_axis_name="c", subcore_axis_name="s")
    return pl.kernel(
        _sc_body,
        out_type=[jax.ShapeDtypeStruct((B,), jnp.float32),
                  jax.ShapeDtypeStruct((NNEG * B,), jnp.float32)],
        mesh=mesh,
        scratch_types=[
            pltpu.VMEM((CB * CTX,), jnp.int32),
            pltpu.VMEM((CB * NU,), jnp.int32),
            pltpu.VMEM((CB * CTX, D), jnp.float32),
            pltpu.VMEM((CB * NU, D), jnp.float32),
            pltpu.VMEM((NU * CB * LANES,), jnp.float32),
            pltpu.VMEM((BPW,), jnp.float32),
            pltpu.VMEM((NNEG * BPW,), jnp.float32),
            pltpu.SemaphoreType.DMA,
        ],
        compiler_params=pltpu.CompilerParams(use_tc_tiling_on_sc=False),
    )


def _tc_body(pos_ref, neg_ref, out_ref):
    p = pos_ref[...]
    n = neg_ref[...]

    def softplus(x):
        return jnp.maximum(x, 0.0) + jnp.log(1.0 + jnp.exp(-jnp.abs(x)))

    lp = jnp.minimum(softplus(-p), 100.0)
    ln = jnp.minimum(softplus(n), 100.0)
    out_ref[0, 0] = jnp.sum(lp) * (1.0 / B) + jnp.sum(ln) * (1.0 / (NNEG * B))


def _tc_loss(pos2d, neg2d):
    return pl.pallas_call(
        _tc_body,
        out_shape=jax.ShapeDtypeStruct((1, 1), jnp.float32),
        out_specs=pl.BlockSpec(memory_space=pltpu.SMEM),
    )(pos2d, neg2d)


def kernel(context, pos, neg, u_weight, v_weight):
    ctx_flat = context.reshape(-1)
    u_idx = jnp.concatenate([pos, neg], axis=1).reshape(-1)
    pos_log, neg_log = _sc_call()(ctx_flat, u_idx, u_weight, v_weight)
    loss = _tc_loss(pos_log.reshape(B // 128, 128),
                    neg_log.reshape(NNEG * B // 128, 128))
    return loss[0, 0]


# speedup vs baseline: 1.0204x; 1.0204x over previous
import functools

import jax
import jax.numpy as jnp
from jax import lax
from jax.experimental import pallas as pl
from jax.experimental.pallas import tpu as pltpu
from jax.experimental.pallas import tpu_sc as plsc

B = 16384
CTX = 20
NNEG = 5
NU = NNEG + 1
D = 64
LANES = 16

NC, NS = 2, 16
NW = NC * NS
BPW = B // NW
CB = 16
NCHUNK = BPW // CB
G = D // LANES


VSLICES = ((0, 128), (128, 128), (256, 64))


def _sc_body(ctx_idx, u_idx, u_w, v_w, pos_out, neg_out,
             ctx_v0, uidx_v0, vrows0, urows0,
             ctx_v1, uidx_v1, vrows1, urows1,
             prodbuf, posbuf, negbuf, sem0, sem1):
    wid = lax.axis_index("s") * NC + lax.axis_index("c")
    base = wid * BPW

    lane = lax.iota(jnp.int32, LANES)
    shuf_idx = [lane ^ (1 << s) for s in range(4)]
    shuf_msk = [(lane & (1 << s)) == 0 for s in range(4)]
    bitrev = [0, 8, 4, 12, 2, 10, 6, 14, 1, 9, 5, 13, 3, 11, 7, 15]

    slots = ((ctx_v0, uidx_v0, vrows0, urows0, sem0),
             (ctx_v1, uidx_v1, vrows1, urows1, sem1))

    def _lane_sums(vecs):
        vs = vecs
        for s in range(3, -1, -1):
            idx, m = shuf_idx[s], shuf_msk[s]
            nxt = []
            for a, b in zip(vs[0::2], vs[1::2]):
                ga = a.at[idx].get(mode="promise_in_bounds")
                gb = b.at[idx].get(mode="promise_in_bounds")
                nxt.append(jnp.where(m, a, gb) + jnp.where(m, ga, b))
            vs = nxt
        return vs[0]

    def _stage_start(c, slot):
        ctx_v, uidx_v, vrows, urows, sem = slot
        b0 = base + c * CB
        pltpu.sync_copy(ctx_idx.at[pl.ds(b0 * CTX, CB * CTX)], ctx_v)
        pltpu.sync_copy(u_idx.at[pl.ds(b0 * NU, CB * NU)], uidx_v)
        cps = []
        for s0, sl in VSLICES:
            cps.append(pltpu.async_copy(
                v_w.at[ctx_v.at[pl.ds(s0, sl)]], vrows.at[pl.ds(s0, sl)], sem))
        cps.append(pltpu.async_copy(u_w.at[uidx_v], urows, sem))
        return cps

    def _drain(slot):
        _, _, vrows, urows, sem = slot
        pltpu.make_async_copy(v_w.at[pl.ds(0, CB * CTX)], vrows, sem).wait()
        pltpu.make_async_copy(v_w.at[pl.ds(0, CB * NU)], urows, sem).wait()

    def _compute(c, slot):
        _, _, vrows, urows, _ = slot
        for i in range(CB):
            vm = []
            for g in range(G):
                acc = vrows[i * CTX, pl.ds(g * LANES, LANES)]
                for j in range(1, CTX):
                    acc = acc + vrows[i * CTX + j, pl.ds(g * LANES, LANES)]
                vm.append(acc)
            for k in range(NU):
                prod = vm[0] * urows[i * NU + k, pl.ds(0, LANES)]
                for g in range(1, G):
                    prod = prod + vm[g] * urows[i * NU + k,
                                                pl.ds(g * LANES, LANES)]
                prodbuf[pl.ds((k * CB + i) * LANES, LANES)] = prod
        for k in range(NU):
            vecs = [prodbuf[pl.ds((k * CB + bitrev[j]) * LANES, LANES)]
                    for j in range(CB)]
            logits = _lane_sums(vecs) * (1.0 / CTX)
            if k == 0:
                posbuf[pl.ds(c * CB, CB)] = logits
            else:
                negbuf[pl.ds((k - 1) * BPW + c * CB, CB)] = logits

    _stage_start(0, slots[0])

    def pair(k, carry):
        cps1 = _stage_start(2 * k + 1, slots[1])
        _drain(slots[0])
        _compute(2 * k, slots[0])

        @pl.when(k < NCHUNK // 2 - 1)
        def _():
            _stage_start(2 * k + 2, slots[0])

        for cp in cps1:
            cp.wait()
        _compute(2 * k + 1, slots[1])
        return carry

    lax.fori_loop(0, NCHUNK // 2, pair, 0)
    pltpu.sync_copy(posbuf, pos_out.at[pl.ds(base, BPW)])
    for k in range(NNEG):
        pltpu.sync_copy(negbuf.at[pl.ds(k * BPW, BPW)],
                        neg_out.at[pl.ds(k * B + base, BPW)])


@functools.cache
def _sc_call():
    mesh = plsc.VectorSubcoreMesh(core_axis_name="c", subcore_axis_name="s")
    return pl.kernel(
        _sc_body,
        out_type=[jax.ShapeDtypeStruct((B,), jnp.float32),
                  jax.ShapeDtypeStruct((NNEG * B,), jnp.float32)],
        mesh=mesh,
        scratch_types=[
            pltpu.VMEM((CB * CTX,), jnp.int32),
            pltpu.VMEM((CB * NU,), jnp.int32),
            pltpu.VMEM((CB * CTX, D), jnp.float32),
            pltpu.VMEM((CB * NU, D), jnp.float32),
            pltpu.VMEM((CB * CTX,), jnp.int32),
            pltpu.VMEM((CB * NU,), jnp.int32),
            pltpu.VMEM((CB * CTX, D), jnp.float32),
            pltpu.VMEM((CB * NU, D), jnp.float32),
            pltpu.VMEM((NU * CB * LANES,), jnp.float32),
            pltpu.VMEM((BPW,), jnp.float32),
            pltpu.VMEM((NNEG * BPW,), jnp.float32),
            pltpu.SemaphoreType.DMA,
            pltpu.SemaphoreType.DMA,
        ],
        compiler_params=pltpu.CompilerParams(use_tc_tiling_on_sc=False),
    )


def _tc_body(pos_ref, neg_ref, out_ref):
    p = pos_ref[...]
    n = neg_ref[...]

    def softplus(x):
        return jnp.maximum(x, 0.0) + jnp.log(1.0 + jnp.exp(-jnp.abs(x)))

    lp = jnp.minimum(softplus(-p), 100.0)
    ln = jnp.minimum(softplus(n), 100.0)
    out_ref[0, 0] = jnp.sum(lp) * (1.0 / B) + jnp.sum(ln) * (1.0 / (NNEG * B))


def _tc_loss(pos2d, neg2d):
    return pl.pallas_call(
        _tc_body,
        out_shape=jax.ShapeDtypeStruct((1, 1), jnp.float32),
        out_specs=pl.BlockSpec(memory_space=pltpu.SMEM),
    )(pos2d, neg2d)


def kernel(context, pos, neg, u_weight, v_weight):
    ctx_flat = context.reshape(-1)
    u_idx = jnp.concatenate([pos, neg], axis=1).reshape(-1)
    pos_log, neg_log = _sc_call()(ctx_flat, u_idx, u_weight, v_weight)
    loss = _tc_loss(pos_log.reshape(B // 128, 128),
                    neg_log.reshape(NNEG * B // 128, 128))
    return loss[0, 0]
